# Initial kernel scaffold; baseline (speedup 1.0000x reference)
import jax
import jax.numpy as jnp
from jax.experimental import pallas as pl


def kernel(x_node, edge_index_rel0, edge_index_rel1, edge_label_index, W_proj1, b_proj1, att_src1_r0, att_dst1_r0, att_src1_r1, att_dst1_r1, kW1, kb1, q1, W_proj2, b_proj2, att_src2_r0, att_dst2_r0, att_src2_r1, att_dst2_r1, kW2, kb2, q2, W_post, b_post):
    raise NotImplementedError("write your pallas kernel here")


# speedup vs baseline: 5.2068x; 5.2068x over previous
import functools
import jax
import jax.numpy as jnp
from jax import lax
from jax.experimental import pallas as pl
from jax.experimental.pallas import tpu as pltpu
from jax.experimental.pallas import tpu_sc as plsc

N_NODES = 10000
N_PAD = 10112
E_EDGES = 160000
E_PAD = 163840
B_LBL_N = 20000
B_PAD = 20480
NC, NS = 2, 16
NW = NC * NS
CHUNK = 128
W128 = 128
F32 = jnp.float32


def _mesh():
    return plsc.VectorSubcoreMesh(core_axis_name="c", subcore_axis_name="s")


def _make_gather4(ep):
    n_chunks = ep // NW // CHUNK

    @functools.partial(
        pl.kernel,
        mesh=_mesh(),
        out_type=[jax.ShapeDtypeStruct((ep, W128), F32) for _ in range(4)],
        scratch_types=[
            pltpu.VMEM((CHUNK,), jnp.int32),
            pltpu.VMEM((CHUNK, W128), F32),
            pltpu.SemaphoreType.DMA,
        ],
    )
    def k(tab, src0, dst0, src1, dst1, gs0, gd0, gs1, gd1, idx_v, rows_v, sem):
        wid = lax.axis_index("s") * NC + lax.axis_index("c")

        def body(j, carry):
            base = wid * (n_chunks * CHUNK) + j * CHUNK
            for idx, go in ((src0, gs0), (dst0, gd0), (src1, gs1), (dst1, gd1)):
                pltpu.sync_copy(idx.at[pl.ds(base, CHUNK)], idx_v)
                pltpu.async_copy(tab.at[idx_v], rows_v, sem).wait()
                pltpu.sync_copy(rows_v, go.at[pl.ds(base, CHUNK)])
            return carry

        lax.fori_loop(0, n_chunks, body, 0)

    return k


def _make_gather_pair(bp):
    n_chunks = bp // NW // CHUNK

    @functools.partial(
        pl.kernel,
        mesh=_mesh(),
        out_type=[jax.ShapeDtypeStruct((bp, W128), F32) for _ in range(2)],
        scratch_types=[
            pltpu.VMEM((CHUNK,), jnp.int32),
            pltpu.VMEM((CHUNK, W128), F32),
            pltpu.SemaphoreType.DMA,
        ],
    )
    def k(tab, i0, i1, f0, f1, idx_v, rows_v, sem):
        wid = lax.axis_index("s") * NC + lax.axis_index("c")

        def body(j, carry):
            base = wid * (n_chunks * CHUNK) + j * CHUNK
            for idx, go in ((i0, f0), (i1, f1)):
                pltpu.sync_copy(idx.at[pl.ds(base, CHUNK)], idx_v)
                pltpu.async_copy(tab.at[idx_v], rows_v, sem).wait()
                pltpu.sync_copy(rows_v, go.at[pl.ds(base, CHUNK)])
            return carry

        lax.fori_loop(0, n_chunks, body, 0)

    return k


def _make_scatter(ep):
    n_chunks = ep // NW // CHUNK
    rows_per_sub = N_PAD // NS

    @functools.partial(
        pl.kernel,
        mesh=_mesh(),
        out_type=jax.ShapeDtypeStruct((NC, N_PAD, W128), F32),
        scratch_types=[
            pltpu.VMEM((CHUNK,), jnp.int32),
            pltpu.VMEM((CHUNK, W128), F32),
            pltpu.VMEM_SHARED((N_PAD, W128), F32),
            pltpu.SemaphoreType.DMA,
        ],
    )
    def k(zeros_hbm, dst, wv, out, idx_v, val_v, acc, sem):
        cid = lax.axis_index("c")
        sid = lax.axis_index("s")
        wid = sid * NC + cid
        rbase = sid * rows_per_sub
        pltpu.sync_copy(zeros_hbm.at[pl.ds(rbase, rows_per_sub)],
                        acc.at[pl.ds(rbase, rows_per_sub)])
        plsc.subcore_barrier()

        def body(j, carry):
            base = wid * (n_chunks * CHUNK) + j * CHUNK
            pltpu.sync_copy(dst.at[pl.ds(base, CHUNK)], idx_v)
            pltpu.sync_copy(wv.at[pl.ds(base, CHUNK)], val_v)
            pltpu.sync_copy(val_v, acc.at[idx_v], add=True)
            return carry

        lax.fori_loop(0, n_chunks, body, 0)
        plsc.subcore_barrier()
        pltpu.sync_copy(acc.at[pl.ds(rbase, rows_per_sub)],
                        out.at[cid, pl.ds(rbase, rows_per_sub)])

    return k


def _proj1_kernel(x_ref, w_ref, b_ref, as0_ref, ad0_ref, as1_ref, ad1_ref,
                  tab_ref):
    h = jnp.dot(x_ref[...], w_ref[...], preferred_element_type=F32, precision=lax.Precision.HIGHEST) + b_ref[...][None, :]
    tab_ref[:, :64] = h
    tab_ref[:, 64:65] = jnp.sum(h * as0_ref[...][None, :], axis=1, keepdims=True)
    tab_ref[:, 65:66] = jnp.sum(h * as1_ref[...][None, :], axis=1, keepdims=True)
    tab_ref[:, 66:67] = jnp.sum(h * ad0_ref[...][None, :], axis=1, keepdims=True)
    tab_ref[:, 67:68] = jnp.sum(h * ad1_ref[...][None, :], axis=1, keepdims=True)
    tab_ref[:, 68:] = jnp.zeros_like(tab_ref[:, 68:])


def _edge_kernel(gs_ref, gd_ref, out_ref, *, d, rel, blk, e_real):
    pid = pl.program_id(0)
    s = gs_ref[:, d + rel:d + rel + 1]
    dv = gd_ref[:, d + 2 + rel:d + 3 + rel]
    a = s + dv
    a = jnp.where(a >= 0, a, 0.2 * a)
    ex = jnp.exp(a)
    eid = pid * blk + lax.broadcasted_iota(jnp.int32, (blk, 1), 0)
    ex = jnp.where(eid < e_real, ex, 0.0)
    out_ref[:, :d] = gs_ref[:, :d] * ex
    out_ref[:, d:d + 1] = ex
    out_ref[:, d + 1:] = jnp.zeros_like(out_ref[:, d + 1:])


def _norm_kernel(acc_ref, o_ref, *, d):
    num = acc_ref[0, :, :d] + acc_ref[1, :, :d]
    den = acc_ref[0, :, d:d + 1] + acc_ref[1, :, d:d + 1]
    o_ref[...] = jax.nn.relu(num / (den + 1e-16))


def _semantic(o0, o1, kw, kb, q, d):
    rmask = (lax.broadcasted_iota(jnp.int32, (N_PAD, 1), 0) < N_NODES).astype(F32)
    t0 = jnp.tanh(jnp.dot(o0, kw, preferred_element_type=F32) + kb[None, :]) * rmask
    t1 = jnp.tanh(jnp.dot(o1, kw, preferred_element_type=F32) + kb[None, :]) * rmask
    k0 = jnp.sum(t0, axis=0, keepdims=True) / N_NODES
    k1 = jnp.sum(t1, axis=0, keepdims=True) / N_NODES
    s0 = jnp.sum(k0 * q[None, :])
    s1 = jnp.sum(k1 * q[None, :])
    m = jnp.maximum(s0, s1)
    e0 = jnp.exp(s0 - m)
    e1 = jnp.exp(s1 - m)
    a0 = e0 / (e0 + e1)
    a1 = e1 / (e0 + e1)
    return jax.nn.relu(a0 * o0 + a1 * o1)


def _combine1_kernel(o0_ref, o1_ref, kw_ref, kb_ref, q_ref,
                     w2_ref, b2_ref, as0_ref, ad0_ref, as1_ref, ad1_ref,
                     tab_ref):
    hr = _semantic(o0_ref[...], o1_ref[...], kw_ref[...], kb_ref[...],
                   q_ref[...], 64)
    h2 = jnp.dot(hr, w2_ref[...], preferred_element_type=F32, precision=lax.Precision.HIGHEST) + b2_ref[...][None, :]
    tab_ref[:, :32] = h2
    tab_ref[:, 32:33] = jnp.sum(h2 * as0_ref[...][None, :], axis=1, keepdims=True)
    tab_ref[:, 33:34] = jnp.sum(h2 * as1_ref[...][None, :], axis=1, keepdims=True)
    tab_ref[:, 34:35] = jnp.sum(h2 * ad0_ref[...][None, :], axis=1, keepdims=True)
    tab_ref[:, 35:36] = jnp.sum(h2 * ad1_ref[...][None, :], axis=1, keepdims=True)
    tab_ref[:, 36:] = jnp.zeros_like(tab_ref[:, 36:])


def _combine2_kernel(o0_ref, o1_ref, kw_ref, kb_ref, q_ref, tab_ref):
    tab_ref[:, :32] = _semantic(o0_ref[...], o1_ref[...], kw_ref[...],
                                kb_ref[...], q_ref[...], 32)
    tab_ref[:, 32:] = jnp.zeros_like(tab_ref[:, 32:])


def _final_kernel(f0_ref, f1_ref, wp_ref, bp_ref, out_ref):
    w = jnp.sum(wp_ref[...], axis=1)
    hh = f0_ref[:, :32] * f1_ref[:, :32]
    out_ref[...] = (jnp.sum(hh * w[None, :], axis=1, keepdims=True)
                    + jnp.sum(bp_ref[...]))


def _pad_idx(a, n):
    return jnp.concatenate([a, jnp.zeros((n - a.shape[0],), jnp.int32)])


def kernel(x_node, edge_index_rel0, edge_index_rel1, edge_label_index,
           W_proj1, b_proj1, att_src1_r0, att_dst1_r0, att_src1_r1, att_dst1_r1,
           kW1, kb1, q1,
           W_proj2, b_proj2, att_src2_r0, att_dst2_r0, att_src2_r1, att_dst2_r1,
           kW2, kb2, q2, W_post, b_post):
    x_pad = jnp.concatenate(
        [x_node, jnp.zeros((N_PAD - N_NODES, x_node.shape[1]), F32)])
    src0 = _pad_idx(edge_index_rel0[0], E_PAD)
    dst0 = _pad_idx(edge_index_rel0[1], E_PAD)
    src1 = _pad_idx(edge_index_rel1[0], E_PAD)
    dst1 = _pad_idx(edge_index_rel1[1], E_PAD)
    eli0 = _pad_idx(edge_label_index[0], B_PAD)
    eli1 = _pad_idx(edge_label_index[1], B_PAD)
    zeros = jnp.zeros((N_PAD, W128), F32)

    nblk = N_PAD // 8
    vec64 = pl.BlockSpec((64,), lambda i: (0,))
    tab1 = pl.pallas_call(
        _proj1_kernel,
        grid=(8,),
        in_specs=[pl.BlockSpec((nblk, 128), lambda i: (i, 0)),
                  pl.BlockSpec((128, 64), lambda i: (0, 0)),
                  vec64, vec64, vec64, vec64, vec64],
        out_specs=pl.BlockSpec((nblk, W128), lambda i: (i, 0)),
        out_shape=jax.ShapeDtypeStruct((N_PAD, W128), F32),
    )(x_pad, W_proj1, b_proj1, att_src1_r0, att_dst1_r0, att_src1_r1,
      att_dst1_r1)

    def run_layer(tab, d):
        gs0, gd0, gs1, gd1 = _make_gather4(E_PAD)(tab, src0, dst0, src1, dst1)
        blk = 4096
        grid = E_PAD // blk

        def edge(gs, gd, rel):
            return pl.pallas_call(
                functools.partial(_edge_kernel, d=d, rel=rel, blk=blk,
                                  e_real=E_EDGES),
                grid=(grid,),
                in_specs=[pl.BlockSpec((blk, W128), lambda i: (i, 0)),
                          pl.BlockSpec((blk, W128), lambda i: (i, 0))],
                out_specs=pl.BlockSpec((blk, W128), lambda i: (i, 0)),
                out_shape=jax.ShapeDtypeStruct((E_PAD, W128), F32),
            )(gs, gd)

        wr0 = edge(gs0, gd0, 0)
        wr1 = edge(gs1, gd1, 1)
        scat = _make_scatter(E_PAD)
        acc0 = scat(zeros, dst0, wr0)
        acc1 = scat(zeros, dst1, wr1)
        norm = pl.pallas_call(
            functools.partial(_norm_kernel, d=d),
            out_shape=jax.ShapeDtypeStruct((N_PAD, d), F32),
        )
        return norm(acc0), norm(acc1)

    o0, o1 = run_layer(tab1, 64)
    tab2 = pl.pallas_call(
        _combine1_kernel,
        out_shape=jax.ShapeDtypeStruct((N_PAD, W128), F32),
    )(o0, o1, kW1, kb1, q1, W_proj2, b_proj2,
      att_src2_r0, att_dst2_r0, att_src2_r1, att_dst2_r1)

    o0b, o1b = run_layer(tab2, 32)
    ftab = pl.pallas_call(
        _combine2_kernel,
        out_shape=jax.ShapeDtypeStruct((N_PAD, W128), F32),
    )(o0b, o1b, kW2, kb2, q2)

    f0, f1 = _make_gather_pair(B_PAD)(ftab, eli0, eli1)
    fblk = B_PAD // 5
    out = pl.pallas_call(
        _final_kernel,
        grid=(5,),
        in_specs=[pl.BlockSpec((fblk, W128), lambda i: (i, 0)),
                  pl.BlockSpec((fblk, W128), lambda i: (i, 0)),
                  pl.BlockSpec((32, 2), lambda i: (0, 0)),
                  pl.BlockSpec((2,), lambda i: (0,))],
        out_specs=pl.BlockSpec((fblk, 1), lambda i: (i, 0)),
        out_shape=jax.ShapeDtypeStruct((B_PAD, 1), F32),
    )(f0, f1, W_post, b_post)
    return out[:B_LBL_N, 0]
